# baseline (speedup 1.0000x reference)
import functools

import jax
import jax.numpy as jnp
from jax import lax
from jax.experimental import pallas as pl
from jax.experimental.pallas import tpu as pltpu
from jax.experimental.pallas import tpu_sc as plsc

D_MODEL = 768
BATCH = 4
SEQ = 2048
B = BATCH * SEQ
NC, NS = 2, 16
NW = NC * NS
BPW = B // NW
CH = 64
NCHUNK = BPW // CH

_mesh = plsc.VectorSubcoreMesh(core_axis_name="c", subcore_axis_name="s")


@functools.partial(
    pl.kernel,
    out_type=jax.ShapeDtypeStruct((B, D_MODEL), jnp.float32),
    mesh=_mesh,
    scratch_types=[
        pltpu.VMEM((NCHUNK, CH), jnp.int32),
        pltpu.VMEM((CH, D_MODEL), jnp.float32),
        pltpu.VMEM((CH, D_MODEL), jnp.float32),
        pltpu.SemaphoreType.DMA,
        pltpu.SemaphoreType.DMA,
        pltpu.SemaphoreType.DMA,
        pltpu.SemaphoreType.DMA,
    ],
)
def _embed_gather(ids_hbm, table_hbm, out_hbm, idx_v, rows0, rows1,
                  gsem0, gsem1, ssem0, ssem1):
    wid = lax.axis_index("s") * NC + lax.axis_index("c")
    base = wid * BPW
    bufs = (rows0, rows1)
    gsems = (gsem0, gsem1)
    ssems = (ssem0, ssem1)
    pltpu.sync_copy(ids_hbm.at[wid], idx_v)

    gathers = [None] * NCHUNK
    scatters = [None] * NCHUNK
    gathers[0] = pltpu.async_copy(table_hbm.at[idx_v.at[0]], bufs[0], gsems[0])
    for c in range(NCHUNK):
        if c + 1 < NCHUNK:
            if c >= 1:
                scatters[c - 1].wait()
            gathers[c + 1] = pltpu.async_copy(
                table_hbm.at[idx_v.at[c + 1]], bufs[(c + 1) % 2],
                gsems[(c + 1) % 2])
        gathers[c].wait()
        scatters[c] = pltpu.async_copy(
            bufs[c % 2], out_hbm.at[pl.ds(base + c * CH, CH)], ssems[c % 2])
    scatters[NCHUNK - 2].wait()
    scatters[NCHUNK - 1].wait()


def kernel(input_ids, embed_tokens_weight):
    ids = input_ids.astype(jnp.int32).reshape(NW, NCHUNK, CH)
    out = _embed_gather(ids, embed_tokens_weight)
    return out.reshape(BATCH, SEQ, D_MODEL)


# speedup vs baseline: 1.0022x; 1.0022x over previous
import functools

import jax
import jax.numpy as jnp
from jax import lax
from jax.experimental import pallas as pl
from jax.experimental.pallas import tpu as pltpu
from jax.experimental.pallas import tpu_sc as plsc

D_MODEL = 768
BATCH = 4
SEQ = 2048
NC, NS = 2, 16
NW = NC * NS
WPB = NW // BATCH
BPW = SEQ // WPB
CH = 64
NCHUNK = BPW // CH

_mesh = plsc.VectorSubcoreMesh(core_axis_name="c", subcore_axis_name="s")


@functools.partial(
    pl.kernel,
    out_type=jax.ShapeDtypeStruct((BATCH, SEQ, D_MODEL), jnp.float32),
    mesh=_mesh,
    scratch_types=[
        pltpu.VMEM((BPW,), jnp.int32),
        pltpu.VMEM((CH, D_MODEL), jnp.float32),
        pltpu.VMEM((CH, D_MODEL), jnp.float32),
        pltpu.SemaphoreType.DMA,
        pltpu.SemaphoreType.DMA,
        pltpu.SemaphoreType.DMA,
        pltpu.SemaphoreType.DMA,
    ],
)
def _embed_gather(ids_hbm, table_hbm, out_hbm, idx_v, rows0, rows1,
                  gsem0, gsem1, ssem0, ssem1):
    wid = lax.axis_index("s") * NC + lax.axis_index("c")
    b = wid // WPB
    col0 = (wid % WPB) * BPW
    bufs = (rows0, rows1)
    gsems = (gsem0, gsem1)
    ssems = (ssem0, ssem1)
    pltpu.sync_copy(ids_hbm.at[b, pl.ds(col0, BPW)], idx_v)

    gathers = [None] * NCHUNK
    scatters = [None] * NCHUNK
    gathers[0] = pltpu.async_copy(
        table_hbm.at[idx_v.at[pl.ds(0, CH)]], bufs[0], gsems[0])
    for c in range(NCHUNK):
        if c + 1 < NCHUNK:
            if c >= 1:
                scatters[c - 1].wait()
            gathers[c + 1] = pltpu.async_copy(
                table_hbm.at[idx_v.at[pl.ds((c + 1) * CH, CH)]],
                bufs[(c + 1) % 2], gsems[(c + 1) % 2])
        gathers[c].wait()
        scatters[c] = pltpu.async_copy(
            bufs[c % 2], out_hbm.at[b, pl.ds(col0 + c * CH, CH)],
            ssems[c % 2])
    scatters[NCHUNK - 2].wait()
    scatters[NCHUNK - 1].wait()


def kernel(input_ids, embed_tokens_weight):
    return _embed_gather(input_ids.astype(jnp.int32), embed_tokens_weight)
